# baseline (speedup 1.0000x reference)
import numpy as np
import jax
from jax.experimental import pallas as pl
from jax.experimental.pallas import tpu as pltpu

_CHUNK = 16
_NBUF = 3
_PRE = 2


def kernel(frames):
    C, T, H, W = frames.shape
    S = T // 4
    idx = [int(v) for v in np.linspace(0, T - 1, S).astype(np.int64)]
    nchunks = T // _CHUNK
    chunk_gather = [
        [(t - k * _CHUNK, j) for j, t in enumerate(idx)
         if k * _CHUNK <= t < (k + 1) * _CHUNK]
        for k in range(nchunks)
    ]

    def body(x_ref, slow_ref, fast_ref, buf, in_sem, out_sem):
        def in_copies(k):
            h = _CHUNK // 2
            return [
                pltpu.make_async_copy(
                    x_ref.at[:, k * _CHUNK:k * _CHUNK + h],
                    buf.at[k % _NBUF, :, 0:h], in_sem.at[k % _NBUF, 0]),
                pltpu.make_async_copy(
                    x_ref.at[:, k * _CHUNK + h:(k + 1) * _CHUNK],
                    buf.at[k % _NBUF, :, h:_CHUNK], in_sem.at[k % _NBUF, 1]),
            ]

        def out_copies(k):
            h = _CHUNK // 2
            cs = [
                pltpu.make_async_copy(
                    buf.at[k % _NBUF, :, 0:h],
                    fast_ref.at[:, k * _CHUNK:k * _CHUNK + h],
                    out_sem.at[k % _NBUF, 0]),
                pltpu.make_async_copy(
                    buf.at[k % _NBUF, :, h:_CHUNK],
                    fast_ref.at[:, k * _CHUNK + h:(k + 1) * _CHUNK],
                    out_sem.at[k % _NBUF, 1]),
            ]
            for off, j in chunk_gather[k]:
                cs.append(pltpu.make_async_copy(
                    buf.at[k % _NBUF, :, off:off + 1],
                    slow_ref.at[:, j:j + 1], out_sem.at[k % _NBUF, 2]))
            return cs

        for k in range(_PRE):
            for c in in_copies(k):
                c.start()
        for k in range(nchunks):
            look = k + _PRE
            if look < nchunks:
                prev = look - _NBUF
                if prev >= 0:
                    for c in out_copies(prev):
                        c.wait()
                for c in in_copies(look):
                    c.start()
            for c in in_copies(k):
                c.wait()
            for c in out_copies(k):
                c.start()
        for k in range(max(0, nchunks - _NBUF), nchunks):
            for c in out_copies(k):
                c.wait()

    slow, fast = pl.pallas_call(
        body,
        in_specs=[pl.BlockSpec(memory_space=pl.ANY)],
        out_specs=[
            pl.BlockSpec(memory_space=pl.ANY),
            pl.BlockSpec(memory_space=pl.ANY),
        ],
        out_shape=[
            jax.ShapeDtypeStruct((C, S, H, W), frames.dtype),
            jax.ShapeDtypeStruct((C, T, H, W), frames.dtype),
        ],
        scratch_shapes=[
            pltpu.VMEM((_NBUF, C, _CHUNK, H, W), frames.dtype),
            pltpu.SemaphoreType.DMA((_NBUF, 2)),
            pltpu.SemaphoreType.DMA((_NBUF, 3)),
        ],
    )(frames)
    return (slow, fast)


# speedup vs baseline: 1.0018x; 1.0018x over previous
import numpy as np
import jax
from jax.experimental import pallas as pl
from jax.experimental.pallas import tpu as pltpu

_CHUNK = 16
_NBUF = 3
_PRE = 2


def kernel(frames):
    C, T, H, W = frames.shape
    S = T // 4
    idx = [int(v) for v in np.linspace(0, T - 1, S).astype(np.int64)]
    nchunks = T // _CHUNK
    chunk_gather = [
        [(t - k * _CHUNK, j) for j, t in enumerate(idx)
         if k * _CHUNK <= t < (k + 1) * _CHUNK]
        for k in range(nchunks)
    ]

    def body(x_ref, slow_ref, fast_ref, buf, in_sem, out_sem):
        def in_copy(k):
            return pltpu.make_async_copy(
                x_ref.at[:, k * _CHUNK:(k + 1) * _CHUNK],
                buf.at[k % _NBUF], in_sem.at[k % _NBUF])

        def out_copies(k):
            h = _CHUNK // 2
            cs = [
                pltpu.make_async_copy(
                    buf.at[k % _NBUF, :, 0:h],
                    fast_ref.at[:, k * _CHUNK:k * _CHUNK + h],
                    out_sem.at[k % _NBUF, 0]),
                pltpu.make_async_copy(
                    buf.at[k % _NBUF, :, h:_CHUNK],
                    fast_ref.at[:, k * _CHUNK + h:(k + 1) * _CHUNK],
                    out_sem.at[k % _NBUF, 1]),
            ]
            for off, j in chunk_gather[k]:
                cs.append(pltpu.make_async_copy(
                    buf.at[k % _NBUF, :, off:off + 1],
                    slow_ref.at[:, j:j + 1], out_sem.at[k % _NBUF, 2]))
            return cs

        for k in range(_PRE):
            in_copy(k).start()
        for k in range(nchunks):
            look = k + _PRE
            if look < nchunks:
                prev = look - _NBUF
                if prev >= 0:
                    for c in out_copies(prev):
                        c.wait()
                in_copy(look).start()
            in_copy(k).wait()
            for c in out_copies(k):
                c.start()
        for k in range(max(0, nchunks - _NBUF), nchunks):
            for c in out_copies(k):
                c.wait()

    slow, fast = pl.pallas_call(
        body,
        in_specs=[pl.BlockSpec(memory_space=pl.ANY)],
        out_specs=[
            pl.BlockSpec(memory_space=pl.ANY),
            pl.BlockSpec(memory_space=pl.ANY),
        ],
        out_shape=[
            jax.ShapeDtypeStruct((C, S, H, W), frames.dtype),
            jax.ShapeDtypeStruct((C, T, H, W), frames.dtype),
        ],
        scratch_shapes=[
            pltpu.VMEM((_NBUF, C, _CHUNK, H, W), frames.dtype),
            pltpu.SemaphoreType.DMA((_NBUF,)),
            pltpu.SemaphoreType.DMA((_NBUF, 3)),
        ],
    )(frames)
    return (slow, fast)


# speedup vs baseline: 1.0199x; 1.0180x over previous
import numpy as np
import jax
from jax.experimental import pallas as pl
from jax.experimental.pallas import tpu as pltpu

_CHUNK = 32
_NBUF = 2
_PRE = 1


def kernel(frames):
    C, T, H, W = frames.shape
    S = T // 4
    idx = [int(v) for v in np.linspace(0, T - 1, S).astype(np.int64)]
    nchunks = T // _CHUNK
    chunk_gather = [
        [(t - k * _CHUNK, j) for j, t in enumerate(idx)
         if k * _CHUNK <= t < (k + 1) * _CHUNK]
        for k in range(nchunks)
    ]

    def body(x_ref, slow_ref, fast_ref, buf, in_sem, out_sem):
        def in_copy(k):
            return pltpu.make_async_copy(
                x_ref.at[:, k * _CHUNK:(k + 1) * _CHUNK],
                buf.at[k % _NBUF], in_sem.at[k % _NBUF])

        def out_copies(k):
            h = _CHUNK // 2
            cs = [
                pltpu.make_async_copy(
                    buf.at[k % _NBUF, :, 0:h],
                    fast_ref.at[:, k * _CHUNK:k * _CHUNK + h],
                    out_sem.at[k % _NBUF, 0]),
                pltpu.make_async_copy(
                    buf.at[k % _NBUF, :, h:_CHUNK],
                    fast_ref.at[:, k * _CHUNK + h:(k + 1) * _CHUNK],
                    out_sem.at[k % _NBUF, 1]),
            ]
            for off, j in chunk_gather[k]:
                cs.append(pltpu.make_async_copy(
                    buf.at[k % _NBUF, :, off:off + 1],
                    slow_ref.at[:, j:j + 1], out_sem.at[k % _NBUF, 2]))
            return cs

        for k in range(_PRE):
            in_copy(k).start()
        for k in range(nchunks):
            look = k + _PRE
            if look < nchunks:
                prev = look - _NBUF
                if prev >= 0:
                    for c in out_copies(prev):
                        c.wait()
                in_copy(look).start()
            in_copy(k).wait()
            for c in out_copies(k):
                c.start()
        for k in range(max(0, nchunks - _NBUF), nchunks):
            for c in out_copies(k):
                c.wait()

    slow, fast = pl.pallas_call(
        body,
        in_specs=[pl.BlockSpec(memory_space=pl.ANY)],
        out_specs=[
            pl.BlockSpec(memory_space=pl.ANY),
            pl.BlockSpec(memory_space=pl.ANY),
        ],
        out_shape=[
            jax.ShapeDtypeStruct((C, S, H, W), frames.dtype),
            jax.ShapeDtypeStruct((C, T, H, W), frames.dtype),
        ],
        scratch_shapes=[
            pltpu.VMEM((_NBUF, C, _CHUNK, H, W), frames.dtype),
            pltpu.SemaphoreType.DMA((_NBUF,)),
            pltpu.SemaphoreType.DMA((_NBUF, 3)),
        ],
    )(frames)
    return (slow, fast)
